# baseline (speedup 1.0000x reference)
import jax
import jax.numpy as jnp
from jax import lax
from jax.experimental import pallas as pl
from jax.experimental.pallas import tpu as pltpu
from jax.experimental.pallas import tpu_sc as plsc

D = 64
L = 3
C = 1024
NC = 2
NS = 16
NW = NC * NS
LANES = 16
K = 160
G = 4
GSHIFT = 2


def _sc_body(ids_hbm, lookup_hbm, sid_hbm, ind_hbm, out_hbm,
             ids_v, cidx_v, iidx_v, offs_v, sidx_v, tmp_v, ind_v, out_v,
             sem_codes, sem_ind, sem_sid):
    n_total = ids_hbm.shape[0]
    per_w = n_total // NW
    n_chunks = per_w // K
    wid = lax.axis_index("s") * NC + lax.axis_index("c")

    def chunk_body(ci, carry):
        base = wid * per_w + ci * K
        pltpu.sync_copy(ids_hbm.at[pl.ds(base, K)], ids_v)
        for c in range(K // LANES):
            s = pl.ds(c * LANES, LANES)
            v = ids_v[s]
            iidx_v[s] = jax.lax.shift_right_logical(v, GSHIFT)
            offs_v[s] = jax.lax.bitwise_and(v, G - 1) * D
            v = v * L
            for l in range(L):
                cidx_v[pl.ds(l * K + c * LANES, LANES)] = v + l
        codes_dma = pltpu.async_copy(lookup_hbm.at[cidx_v], sidx_v, sem_codes)
        ind_dma = pltpu.async_copy(ind_hbm.at[iidx_v], ind_v, sem_ind)
        codes_dma.wait()
        for l in range(L):
            off = jnp.int32(l * C + 1)
            for c in range(K // LANES):
                s = pl.ds(l * K + c * LANES, LANES)
                sidx_v[s] = sidx_v[s] + off
        pltpu.async_copy(sid_hbm.at[sidx_v], tmp_v, sem_sid).wait()
        ind_dma.wait()

        def add_body(i, carry2):
            off = offs_v[pl.ds(i, LANES)][0]
            for c in range(D // LANES):
                s = pl.ds(c * LANES, LANES)
                out_v[i, s] = (ind_v[i, pl.ds(off + c * LANES, LANES)]
                               + tmp_v[i, s] + tmp_v[K + i, s]
                               + tmp_v[2 * K + i, s])
            return carry2

        lax.fori_loop(0, K, add_body, 0)
        pltpu.sync_copy(out_v, out_hbm.at[pl.ds(base, K)])
        return carry

    lax.fori_loop(0, n_chunks, chunk_body, 0)


def _impl(ids, lookup_flat, sid_table, ind_table):
    n = ids.shape[0]
    mesh = plsc.VectorSubcoreMesh(core_axis_name="c", subcore_axis_name="s")
    fn = pl.kernel(
        _sc_body,
        out_type=jax.ShapeDtypeStruct((n, D), jnp.float32),
        mesh=mesh,
        compiler_params=pltpu.CompilerParams(use_tc_tiling_on_sc=False),
        scratch_types=[
            pltpu.VMEM((K,), jnp.int32),
            pltpu.VMEM((L * K,), jnp.int32),
            pltpu.VMEM((K,), jnp.int32),
            pltpu.VMEM((K + LANES,), jnp.int32),
            pltpu.VMEM((L * K,), jnp.int32),
            pltpu.VMEM((L * K, D), jnp.float32),
            pltpu.VMEM((K, G * D), jnp.float32),
            pltpu.VMEM((K, D), jnp.float32),
            pltpu.SemaphoreType.DMA,
            pltpu.SemaphoreType.DMA,
            pltpu.SemaphoreType.DMA,
        ],
    )
    return fn(ids, lookup_flat, sid_table, ind_table)


def kernel(item_ids, lookup, codebook, sid_table, ind_table):
    b, t = item_ids.shape
    ids = item_ids.reshape(-1)
    lookup_flat = lookup.reshape(-1)
    nrow = ind_table.shape[0]
    pad_rows = (-nrow) % G
    ind2 = jnp.pad(ind_table, ((0, pad_rows), (0, 0))).reshape(
        (nrow + pad_rows) // G, G * D)
    out = _impl(ids, lookup_flat, sid_table, ind2)
    return out.reshape(b, t, D)


# speedup vs baseline: 1.1728x; 1.1728x over previous
import jax
import jax.numpy as jnp
from jax import lax
from jax.experimental import pallas as pl
from jax.experimental.pallas import tpu as pltpu
from jax.experimental.pallas import tpu_sc as plsc
from jax.experimental import layout as jex_layout

D = 64
L = 3
C = 1024
NC = 2
NS = 16
NW = NC * NS
LANES = 16
K = 256
DP = 128


def _sc_body(ids_hbm, lookup_hbm, sid_hbm, ind_hbm, out_hbm,
             ids_v, cidx_v, sidx_v, tmp_v, ind_v, out_v,
             sem_codes, sem_ind, sem_sid):
    n_total = ids_hbm.shape[0]
    per_w = n_total // NW
    n_chunks = per_w // K
    wid = lax.axis_index("s") * NC + lax.axis_index("c")

    def chunk_body(ci, carry):
        base = wid * per_w + ci * K
        pltpu.sync_copy(ids_hbm.at[pl.ds(base, K)], ids_v)
        ind_dma = pltpu.async_copy(ind_hbm.at[ids_v], ind_v, sem_ind)
        for c in range(K // LANES):
            s = pl.ds(c * LANES, LANES)
            v = ids_v[s] * L
            for l in range(L):
                cidx_v[pl.ds(l * K + c * LANES, LANES)] = v + l
        pltpu.async_copy(lookup_hbm.at[cidx_v], sidx_v, sem_codes).wait()
        for l in range(L):
            off = jnp.int32(l * C + 1)
            for c in range(K // LANES):
                s = pl.ds(l * K + c * LANES, LANES)
                sidx_v[s] = sidx_v[s] + off
        pltpu.async_copy(sid_hbm.at[sidx_v], tmp_v, sem_sid).wait()
        ind_dma.wait()

        def add_body(i, carry2):
            for c in range(D // LANES):
                s = pl.ds(c * LANES, LANES)
                out_v[i, s] = (ind_v[i, s] + tmp_v[i, s]
                               + tmp_v[K + i, s] + tmp_v[2 * K + i, s])
            return carry2

        lax.fori_loop(0, K, add_body, 0)
        pltpu.sync_copy(out_v, out_hbm.at[pl.ds(base, K)])
        return carry

    lax.fori_loop(0, n_chunks, chunk_body, 0)


def _impl(ids, lookup_flat, sid_table, ind_table):
    n = ids.shape[0]
    mesh = plsc.VectorSubcoreMesh(core_axis_name="c", subcore_axis_name="s")
    fn = pl.kernel(
        _sc_body,
        out_type=jax.ShapeDtypeStruct((n, D), jnp.float32),
        mesh=mesh,
        compiler_params=pltpu.CompilerParams(use_tc_tiling_on_sc=False),
        scratch_types=[
            pltpu.VMEM((K,), jnp.int32),
            pltpu.VMEM((L * K,), jnp.int32),
            pltpu.VMEM((L * K,), jnp.int32),
            pltpu.VMEM((L * K, D), jnp.float32),
            pltpu.VMEM((K, DP), jnp.float32),
            pltpu.VMEM((K, D), jnp.float32),
            pltpu.SemaphoreType.DMA,
            pltpu.SemaphoreType.DMA,
            pltpu.SemaphoreType.DMA,
        ],
    )
    return fn(ids, lookup_flat, sid_table, ind_table)


def kernel(item_ids, lookup, codebook, sid_table, ind_table):
    b, t = item_ids.shape
    ids = item_ids.reshape(-1)
    lookup_flat = lookup.reshape(-1)
    ind_pad = jnp.pad(ind_table, ((0, 0), (0, DP - D)))
    ind_pad = jex_layout.with_layout_constraint(
        ind_pad, jex_layout.Layout(major_to_minor=(1, 0), tiling=()))
    out = _impl(ids, lookup_flat, sid_table, ind_pad)
    return out.reshape(b, t, D)


# speedup vs baseline: 1.2117x; 1.0331x over previous
import jax
import jax.numpy as jnp
from jax import lax
from jax.experimental import pallas as pl
from jax.experimental.pallas import tpu as pltpu
from jax.experimental.pallas import tpu_sc as plsc

D = 64
L = 3
C = 1024
NC = 2
NS = 16
NW = NC * NS
LANES = 16
KA = 1024
KB = 128
DP = 128


def _idx_body(ids_hbm, lookup_hbm, sidx_hbm, ids_v, cidx_v, sidx_v, sem):
    n_total = ids_hbm.shape[0]
    per_w = n_total // NW
    n_chunks = per_w // KA
    wid = lax.axis_index("s") * NC + lax.axis_index("c")

    def chunk_body(ci, carry):
        base = wid * per_w + ci * KA
        pltpu.sync_copy(ids_hbm.at[pl.ds(base, KA)], ids_v)
        for c in range(KA // LANES):
            v = ids_v[pl.ds(c * LANES, LANES)] * L
            for l in range(L):
                cidx_v[pl.ds(l * KA + c * LANES, LANES)] = v + l
        pltpu.async_copy(lookup_hbm.at[cidx_v], sidx_v, sem).wait()
        for l in range(L):
            off = jnp.int32(l * C + 1)
            for c in range(KA // LANES):
                s = pl.ds(l * KA + c * LANES, LANES)
                sidx_v[s] = sidx_v[s] + off
            pltpu.sync_copy(sidx_v.at[pl.ds(l * KA, KA)],
                            sidx_hbm.at[pl.ds(l * n_total + base, KA)])
        return carry

    lax.fori_loop(0, n_chunks, chunk_body, 0)


def _gather_body(ids_hbm, sidx_hbm, sid_hbm, ind_hbm, out_hbm,
                 ids_v, sidx_v, tmp_v, ind_v, out_v, sem_ind, sem_sid):
    n_total = ids_hbm.shape[0]
    per_w = n_total // NW
    n_chunks = per_w // KB
    wid = lax.axis_index("s") * NC + lax.axis_index("c")

    def chunk_body(ci, carry):
        base = wid * per_w + ci * KB
        pltpu.sync_copy(ids_hbm.at[pl.ds(base, KB)], ids_v)
        ind_dma = pltpu.async_copy(ind_hbm.at[ids_v], ind_v, sem_ind)
        for l in range(L):
            pltpu.sync_copy(sidx_hbm.at[pl.ds(l * n_total + base, KB)],
                            sidx_v.at[pl.ds(l * KB, KB)])
        pltpu.async_copy(sid_hbm.at[sidx_v], tmp_v, sem_sid).wait()
        ind_dma.wait()

        def add_body(i, carry2):
            for c in range(D // LANES):
                s = pl.ds(c * LANES, LANES)
                out_v[i, s] = (ind_v[i, s] + tmp_v[i, s]
                               + tmp_v[KB + i, s] + tmp_v[2 * KB + i, s])
            return carry2

        lax.fori_loop(0, KB, add_body, 0)
        pltpu.sync_copy(out_v, out_hbm.at[pl.ds(base, KB)])
        return carry

    lax.fori_loop(0, n_chunks, chunk_body, 0)


def _impl(ids, lookup_flat, sid_pad, ind_pad):
    n = ids.shape[0]
    mesh = plsc.VectorSubcoreMesh(core_axis_name="c", subcore_axis_name="s")
    sidx = pl.kernel(
        _idx_body,
        out_type=jax.ShapeDtypeStruct((L * n,), jnp.int32),
        mesh=mesh,
        compiler_params=pltpu.CompilerParams(use_tc_tiling_on_sc=False),
        scratch_types=[
            pltpu.VMEM((KA,), jnp.int32),
            pltpu.VMEM((L * KA,), jnp.int32),
            pltpu.VMEM((L * KA,), jnp.int32),
            pltpu.SemaphoreType.DMA,
        ],
    )(ids, lookup_flat)

    out = pl.kernel(
        _gather_body,
        out_type=jax.ShapeDtypeStruct((n, DP), jnp.float32),
        mesh=mesh,
        compiler_params=pltpu.CompilerParams(use_tc_tiling_on_sc=True),
        scratch_types=[
            pltpu.VMEM((KB,), jnp.int32),
            pltpu.VMEM((L * KB,), jnp.int32),
            pltpu.VMEM((L * KB, DP), jnp.float32),
            pltpu.VMEM((KB, DP), jnp.float32),
            pltpu.VMEM((KB, DP), jnp.float32),
            pltpu.SemaphoreType.DMA,
            pltpu.SemaphoreType.DMA,
        ],
    )(ids, sidx, sid_pad, ind_pad)
    return out


def kernel(item_ids, lookup, codebook, sid_table, ind_table):
    b, t = item_ids.shape
    ids = item_ids.reshape(-1)
    lookup_flat = lookup.reshape(-1)
    sid_pad = jnp.pad(sid_table, ((0, 0), (0, DP - D)))
    ind_pad = jnp.pad(ind_table, ((0, 0), (0, DP - D)))
    out = _impl(ids, lookup_flat, sid_pad, ind_pad)
    return out[:, :D].reshape(b, t, D)
